# Initial kernel scaffold; baseline (speedup 1.0000x reference)
import jax
import jax.numpy as jnp
from jax.experimental import pallas as pl


def kernel(p2, p3, p4, p5, w0, b0, w1, b1, w2, b2, w3, b3, wc, bc, wb, bb):
    raise NotImplementedError("write your pallas kernel here")


# speedup vs baseline: 1.5929x; 1.5929x over previous
import jax
import jax.numpy as jnp
from jax import lax
from jax.experimental import pallas as pl
from jax.experimental.pallas import tpu as pltpu

C = 256
_LEVELS = ((64, 64), (32, 32), (16, 16), (8, 8))
_NCHUNKS = (8, 2, 1, 1)


def _align8(n):
    return (n + 7) // 8 * 8


def _geom(H, W):
    Wp = W + 2
    N = H * Wp
    P = _align8(Wp + 1)
    M = _align8(P + N + Wp + 1)
    return Wp, N, P, M


def _conv_chunks(src, dst, w_slice, bias, Wp, N, P, nchunks, relu_mask, cout):
    chunk = N // nchunks
    for i in range(nchunks):
        r0 = i * chunk
        acc = jnp.zeros((chunk, cout), jnp.float32)
        for ky in range(3):
            for kx in range(3):
                s = P + r0 + (ky - 1) * Wp + (kx - 1)
                xs = src[pl.ds(s, chunk), :]
                acc = acc + jnp.dot(xs, w_slice(ky * 3 + kx),
                                    preferred_element_type=jnp.float32)
        y = acc + bias
        if relu_mask:
            y = jnp.maximum(y, 0.0)
            col = (r0 + lax.broadcasted_iota(jnp.int32, (chunk, cout), 0)) % Wp
            y = jnp.where((col > 0) & (col < Wp - 1), y, 0.0)
            dst[pl.ds(P + r0, chunk), :] = y
        else:
            dst[pl.ds(r0, chunk), :] = y


def _body(x2, x3, x4, x5, wm, wh, bm, bh, o2, o3, o4, o5, *scratch):
    xs = (x2, x3, x4, x5)
    outs = (o2, o3, o4, o5)
    for l, (H, W) in enumerate(_LEVELS):
        Wp, N, P, M = _geom(H, W)
        A, B = scratch[2 * l], scratch[2 * l + 1]
        for buf in (A, B):
            buf[pl.ds(0, P), :] = jnp.zeros((P, C), jnp.float32)
            buf[pl.ds(P + N, M - P - N), :] = jnp.zeros((M - P - N, C),
                                                        jnp.float32)
        seq = (xs[l], A, B, A, B)
        for layer in range(4):
            bias = bm[layer]
            _conv_chunks(seq[layer], seq[layer + 1],
                         lambda t, layer=layer: wm[layer, pl.ds(t * C, C), :],
                         bias, Wp, N, P, _NCHUNKS[l], True, C)
        _conv_chunks(B, outs[l],
                     lambda t: wh[pl.ds(t * C, C), :],
                     bh[0:1, :], Wp, N, P, _NCHUNKS[l], False, 16)


def kernel(p2, p3, p4, p5, w0, b0, w1, b1, w2, b2, w3, b3, wc, bc, wb, bb):
    xs = []
    for x, (H, W) in zip((p2, p3, p4, p5), _LEVELS):
        Wp, N, P, M = _geom(H, W)
        t = jnp.transpose(x[0], (1, 2, 0))
        t = jnp.pad(t, ((0, 0), (1, 1), (0, 0)))
        t = t.reshape(N, C)
        t = jnp.pad(t, ((P, M - P - N), (0, 0)))
        xs.append(t)
    wm = jnp.stack([w.transpose(2, 3, 1, 0).reshape(9 * C, C)
                    for w in (w0, w1, w2, w3)])
    whc = jnp.concatenate([wc, wb], axis=0)
    wh = whc.transpose(2, 3, 1, 0).reshape(9 * C, 15)
    wh = jnp.pad(wh, ((0, 0), (0, 1)))
    bm = jnp.stack([b.reshape(1, C) for b in (b0, b1, b2, b3)])
    bh = jnp.pad(jnp.concatenate([bc, bb]), (0, 1)).reshape(1, 16)

    out_shape = tuple(jax.ShapeDtypeStruct((H * (W + 2), 16), jnp.float32)
                      for H, W in _LEVELS)
    scratch = []
    for H, W in _LEVELS:
        _, _, _, M = _geom(H, W)
        scratch += [pltpu.VMEM((M, C), jnp.float32),
                    pltpu.VMEM((M, C), jnp.float32)]

    outs = pl.pallas_call(
        _body,
        out_shape=out_shape,
        scratch_shapes=scratch,
    )(*xs, wm, wh, bm, bh)

    results = []
    for o, (H, W) in zip(outs, _LEVELS):
        Wp = W + 2
        y = o.reshape(H, Wp, 16)[:, 1:W + 1, :15]
        y = jnp.transpose(y, (2, 0, 1))
        results.append(y[:3].reshape(1, 3, 1, H, W))
        results.append(y[3:].reshape(1, 3, 4, H, W))
    return tuple(results)


# speedup vs baseline: 1.6969x; 1.0653x over previous
import jax
import jax.numpy as jnp
from jax import lax
from jax.experimental import pallas as pl
from jax.experimental.pallas import tpu as pltpu

C = 256
_ACT_DT = jnp.bfloat16
_LEVELS = ((64, 64), (32, 32), (16, 16), (8, 8))
_NCHUNKS = (8, 2, 1, 1)


def _align8(n):
    return (n + 7) // 8 * 8


def _geom(H, W):
    Wp = W + 2
    N = H * Wp
    P = _align8(Wp + 1)
    M = _align8(P + N + Wp + 1)
    return Wp, N, P, M


def _conv_chunks(src, dst, w_slice, bias, Wp, N, P, nchunks, relu_mask, cout):
    chunk = N // nchunks
    for i in range(nchunks):
        r0 = i * chunk
        acc = jnp.zeros((chunk, cout), jnp.float32)
        for ky in range(3):
            for kx in range(3):
                s = P + r0 + (ky - 1) * Wp + (kx - 1)
                xs = src[pl.ds(s, chunk), :]
                acc = acc + jnp.dot(xs, w_slice(ky * 3 + kx),
                                    preferred_element_type=jnp.float32)
        y = acc + bias
        if relu_mask:
            y = jnp.maximum(y, 0.0)
            col = (r0 + lax.broadcasted_iota(jnp.int32, (chunk, cout), 0)) % Wp
            y = jnp.where((col > 0) & (col < Wp - 1), y, 0.0)
            dst[pl.ds(P + r0, chunk), :] = y.astype(dst.dtype)
        else:
            dst[pl.ds(r0, chunk), :] = y


def _body(x2, x3, x4, x5, wm, wh, bm, bh, o2, o3, o4, o5, *scratch):
    xs = (x2, x3, x4, x5)
    outs = (o2, o3, o4, o5)
    for l, (H, W) in enumerate(_LEVELS):
        Wp, N, P, M = _geom(H, W)
        A, B = scratch[2 * l], scratch[2 * l + 1]
        for buf in (A, B):
            buf[pl.ds(0, P), :] = jnp.zeros((P, C), buf.dtype)
            buf[pl.ds(P + N, M - P - N), :] = jnp.zeros((M - P - N, C),
                                                        buf.dtype)
        seq = (xs[l], A, B, A, B)
        for layer in range(4):
            bias = bm[layer]
            _conv_chunks(seq[layer], seq[layer + 1],
                         lambda t, layer=layer: wm[layer, pl.ds(t * C, C), :],
                         bias, Wp, N, P, _NCHUNKS[l], True, C)
        _conv_chunks(B, outs[l],
                     lambda t: wh[pl.ds(t * C, C), :],
                     bh[0:1, :], Wp, N, P, _NCHUNKS[l], False, 16)


def kernel(p2, p3, p4, p5, w0, b0, w1, b1, w2, b2, w3, b3, wc, bc, wb, bb):
    xs = []
    for x, (H, W) in zip((p2, p3, p4, p5), _LEVELS):
        Wp, N, P, M = _geom(H, W)
        t = jnp.transpose(x[0], (1, 2, 0))
        t = jnp.pad(t, ((0, 0), (1, 1), (0, 0)))
        t = t.reshape(N, C)
        t = jnp.pad(t, ((P, M - P - N), (0, 0)))
        xs.append(t.astype(_ACT_DT))
    wm = jnp.stack([w.transpose(2, 3, 1, 0).reshape(9 * C, C)
                    for w in (w0, w1, w2, w3)]).astype(_ACT_DT)
    whc = jnp.concatenate([wc, wb], axis=0)
    wh = whc.transpose(2, 3, 1, 0).reshape(9 * C, 15)
    wh = jnp.pad(wh, ((0, 0), (0, 1))).astype(_ACT_DT)
    bm = jnp.stack([b.reshape(1, C) for b in (b0, b1, b2, b3)])
    bh = jnp.pad(jnp.concatenate([bc, bb]), (0, 1)).reshape(1, 16)

    out_shape = tuple(jax.ShapeDtypeStruct((H * (W + 2), 16), jnp.float32)
                      for H, W in _LEVELS)
    scratch = []
    for H, W in _LEVELS:
        _, _, _, M = _geom(H, W)
        scratch += [pltpu.VMEM((M, C), _ACT_DT),
                    pltpu.VMEM((M, C), _ACT_DT)]

    outs = pl.pallas_call(
        _body,
        out_shape=out_shape,
        scratch_shapes=scratch,
    )(*xs, wm, wh, bm, bh)

    results = []
    for o, (H, W) in zip(outs, _LEVELS):
        Wp = W + 2
        y = o.reshape(H, Wp, 16)[:, 1:W + 1, :15]
        y = jnp.transpose(y, (2, 0, 1))
        results.append(y[:3].reshape(1, 3, 1, H, W))
        results.append(y[3:].reshape(1, 3, 4, H, W))
    return tuple(results)


# speedup vs baseline: 1.6976x; 1.0004x over previous
import jax
import jax.numpy as jnp
from jax import lax
from jax.experimental import pallas as pl
from jax.experimental.pallas import tpu as pltpu

C = 256
_ACT_DT = jnp.float32
_MM_DT = jnp.bfloat16
_TILE = 8
_LEVELS = ((64, 64), (32, 32), (16, 16), (8, 8))
_NCHUNKS = (8, 2, 1, 1)


def _align(n, a):
    return (n + a - 1) // a * a


def _geom(H, W):
    Wp = _align(W + 2, _TILE)
    N = H * Wp
    P = _align(Wp + 1, _TILE)
    M = _align(P + N + Wp + 1, _TILE)
    return Wp, N, P, M


def _conv_chunks(src, dst, w_slice, bias, H, W, nchunks, relu_mask, cout):
    Wp, N, P, _ = _geom(H, W)
    chunk = N // nchunks
    for i in range(nchunks):
        r0 = i * chunk
        acc = jnp.zeros((chunk, cout), jnp.float32)
        for ky in range(3):
            for kx in range(3):
                s = P + r0 + (ky - 1) * Wp + (kx - 1)
                xs = src[pl.ds(s, chunk), :].astype(_MM_DT)
                acc = acc + jnp.dot(xs, w_slice(ky * 3 + kx),
                                    preferred_element_type=jnp.float32)
        y = acc + bias
        if relu_mask:
            y = jnp.maximum(y, 0.0)
            col = (r0 + lax.broadcasted_iota(jnp.int32, (chunk, cout), 0)) % Wp
            y = jnp.where((col > 0) & (col < W + 1), y, 0.0)
            dst[pl.ds(P + r0, chunk), :] = y.astype(dst.dtype)
        else:
            dst[pl.ds(r0, chunk), :] = y


def _body(x2, x3, x4, x5, wm, wh, bm, bh, o2, o3, o4, o5, *scratch):
    xs = (x2, x3, x4, x5)
    outs = (o2, o3, o4, o5)
    for l, (H, W) in enumerate(_LEVELS):
        Wp, N, P, M = _geom(H, W)
        A, B = scratch[2 * l], scratch[2 * l + 1]
        for buf in (A, B):
            buf[pl.ds(0, P), :] = jnp.zeros((P, C), buf.dtype)
            buf[pl.ds(P + N, M - P - N), :] = jnp.zeros((M - P - N, C),
                                                        buf.dtype)
        seq = (xs[l], A, B, A, B)
        for layer in range(4):
            bias = bm[layer]
            _conv_chunks(seq[layer], seq[layer + 1],
                         lambda t, layer=layer: wm[layer, pl.ds(t * C, C), :],
                         bias, H, W, _NCHUNKS[l], True, C)
        _conv_chunks(B, outs[l],
                     lambda t: wh[pl.ds(t * C, C), :],
                     bh[0:1, :], H, W, _NCHUNKS[l], False, 16)


def kernel(p2, p3, p4, p5, w0, b0, w1, b1, w2, b2, w3, b3, wc, bc, wb, bb):
    xs = []
    for x, (H, W) in zip((p2, p3, p4, p5), _LEVELS):
        Wp, N, P, M = _geom(H, W)
        t = jnp.transpose(x[0], (1, 2, 0))
        t = jnp.pad(t, ((0, 0), (1, Wp - W - 1), (0, 0)))
        t = t.reshape(N, C)
        t = jnp.pad(t, ((P, M - P - N), (0, 0)))
        xs.append(t.astype(_ACT_DT))
    wm = jnp.stack([w.transpose(2, 3, 1, 0).reshape(9 * C, C)
                    for w in (w0, w1, w2, w3)]).astype(_MM_DT)
    whc = jnp.concatenate([wc, wb], axis=0)
    wh = whc.transpose(2, 3, 1, 0).reshape(9 * C, 15)
    wh = jnp.pad(wh, ((0, 0), (0, 1))).astype(_MM_DT)
    bm = jnp.stack([b.reshape(1, C) for b in (b0, b1, b2, b3)])
    bh = jnp.pad(jnp.concatenate([bc, bb]), (0, 1)).reshape(1, 16)

    out_shape = tuple(jax.ShapeDtypeStruct((_geom(H, W)[1], 16), jnp.float32)
                      for H, W in _LEVELS)
    scratch = []
    for H, W in _LEVELS:
        _, _, _, M = _geom(H, W)
        scratch += [pltpu.VMEM((M, C), _ACT_DT),
                    pltpu.VMEM((M, C), _ACT_DT)]

    outs = pl.pallas_call(
        _body,
        out_shape=out_shape,
        scratch_shapes=scratch,
    )(*xs, wm, wh, bm, bh)

    results = []
    for o, (H, W) in zip(outs, _LEVELS):
        Wp = _geom(H, W)[0]
        y = o.reshape(H, Wp, 16)[:, 1:W + 1, :15]
        y = jnp.transpose(y, (2, 0, 1))
        results.append(y[:3].reshape(1, 3, 1, H, W))
        results.append(y[3:].reshape(1, 3, 4, H, W))
    return tuple(results)
